# baseline (speedup 1.0000x reference)
import functools
import math

import jax
import jax.numpy as jnp
from jax import lax
from jax.experimental import pallas as pl
from jax.experimental.pallas import tpu as pltpu

_ROW_TILE = 256
_WIN = 768


def _softmax_av(q, seg_c, kw, vw, seg_w):
    s = lax.dot_general(q.astype(jnp.bfloat16), kw,
                        (((1,), (1,)), ((), ())),
                        preferred_element_type=jnp.float32)
    mask = seg_c == seg_w
    s = jnp.where(mask, s, -jnp.inf)
    m = jnp.max(s, axis=1, keepdims=True)
    p = jnp.exp(s - m)
    denom = jnp.sum(p, axis=1, keepdims=True)
    return jnp.dot(p.astype(jnp.bfloat16), vw,
                   preferred_element_type=jnp.float32) * (1.0 / denom)


def _atten_pool_kernel(meta_ref,
                       x_full_ref, x_tile_ref, segc_ref, segr_ref, segw_ref,
                       wq_ref, bq_ref, wk_ref, bk_ref, wv_ref, bv_ref,
                       ws_ref, bs_ref,
                       out_ref, k_ref, v_ref, o_ref, *, num_segments, scale):
    i = pl.program_id(0)

    @pl.when(i == 0)
    def _init():
        x_full = x_full_ref[:]
        k = jnp.dot(x_full, wk_ref[:],
                    preferred_element_type=jnp.float32) + bk_ref[:]
        v = jnp.dot(x_full, wv_ref[:],
                    preferred_element_type=jnp.float32) + bv_ref[:]
        k_ref[:] = k.astype(jnp.bfloat16)
        v_ref[:] = v.astype(jnp.bfloat16)
        out_ref[:] = jnp.full_like(out_ref, -jnp.inf)

    s0 = pl.multiple_of(meta_ref[0, i], 128)
    fits = meta_ref[1, i] != 0
    first = meta_ref[2, i]
    last = meta_ref[3, i]

    x_t = x_tile_ref[:]
    q = (jnp.dot(x_t, wq_ref[:],
                 preferred_element_type=jnp.float32) + bq_ref[:]) * scale
    seg_c = segc_ref[0]

    @pl.when(fits)
    def _windowed():
        kw = k_ref[pl.ds(s0, _WIN), :]
        vw = v_ref[pl.ds(s0, _WIN), :]
        o_ref[:] = _softmax_av(q, seg_c, kw, vw, segw_ref[0])

    @pl.when(jnp.logical_not(fits))
    def _full():
        o_ref[:] = _softmax_av(q, seg_c, k_ref[:], v_ref[:], segr_ref[:])

    o = o_ref[:] + jnp.dot(x_t, ws_ref[:],
                           preferred_element_type=jnp.float32) + bs_ref[:]

    for b in range(num_segments):
        @pl.when((b >= first) & (b <= last))
        def _pool():
            mb = seg_c == b
            pb = jnp.max(jnp.where(mb, o, -jnp.inf), axis=0,
                         keepdims=True)
            out_ref[b:b + 1, :] = jnp.maximum(out_ref[b:b + 1, :], pb)


def kernel(x, subgbatch, Wq, bq, Wk, bk, Wv, bv, Wskip, bskip):
    n, d = x.shape
    c = Wq.shape[1]
    num_segments = 16
    t = _ROW_TILE
    num_tiles = n // t
    seg = subgbatch.astype(jnp.int32)
    segc = seg.reshape(num_tiles, t, 1)
    segr = seg.reshape(1, n)

    seg2d = seg.reshape(num_tiles, t)
    firsts = seg2d[:, 0]
    lasts = seg2d[:, -1]
    lo = jnp.sum(seg[None, :] < firsts[:, None], axis=1)
    hi = jnp.sum(seg[None, :] <= lasts[:, None], axis=1)
    s0 = jnp.minimum((lo // 128) * 128, n - _WIN)
    fits = ((hi - s0) <= _WIN).astype(jnp.int32)
    meta = jnp.stack([s0.astype(jnp.int32), fits,
                      firsts, lasts], axis=0)
    col_ix = s0[:, None] + jnp.arange(_WIN)[None, :]
    segw = jnp.take(seg, col_ix, axis=0).reshape(num_tiles, 1, _WIN)

    grid_spec = pltpu.PrefetchScalarGridSpec(
        num_scalar_prefetch=1,
        grid=(num_tiles,),
        in_specs=[
            pl.BlockSpec((n, d), lambda i, m: (0, 0)),
            pl.BlockSpec((t, d), lambda i, m: (i, 0)),
            pl.BlockSpec((1, t, 1), lambda i, m: (i, 0, 0)),
            pl.BlockSpec((1, n), lambda i, m: (0, 0)),
            pl.BlockSpec((1, 1, _WIN), lambda i, m: (i, 0, 0)),
            pl.BlockSpec((d, c), lambda i, m: (0, 0)),
            pl.BlockSpec((1, c), lambda i, m: (0, 0)),
            pl.BlockSpec((d, c), lambda i, m: (0, 0)),
            pl.BlockSpec((1, c), lambda i, m: (0, 0)),
            pl.BlockSpec((d, c), lambda i, m: (0, 0)),
            pl.BlockSpec((1, c), lambda i, m: (0, 0)),
            pl.BlockSpec((d, c), lambda i, m: (0, 0)),
            pl.BlockSpec((1, c), lambda i, m: (0, 0)),
        ],
        out_specs=pl.BlockSpec((num_segments, c), lambda i, m: (0, 0)),
        scratch_shapes=[
            pltpu.VMEM((n, c), jnp.bfloat16),
            pltpu.VMEM((n, c), jnp.bfloat16),
            pltpu.VMEM((t, c), jnp.float32),
        ],
    )

    fn = pl.pallas_call(
        functools.partial(_atten_pool_kernel, num_segments=num_segments,
                          scale=1.0 / math.sqrt(c)),
        grid_spec=grid_spec,
        out_shape=jax.ShapeDtypeStruct((num_segments, c), jnp.float32),
    )
    return fn(meta, x, x, segc, segr, segw,
              Wq, bq.reshape(1, c), Wk, bk.reshape(1, c),
              Wv, bv.reshape(1, c), Wskip, bskip.reshape(1, c))


# speedup vs baseline: 3.3925x; 3.3925x over previous
import functools
import math

import jax
import jax.numpy as jnp
from jax import lax
from jax.experimental import pallas as pl
from jax.experimental.pallas import tpu as pltpu

_ROW_TILE = 256


def _atten_pool_kernel(x_full_ref, x_tile_ref, segc_ref, segr_ref,
                       wq_ref, bq_ref, wk_ref, bk_ref, wv_ref, bv_ref,
                       ws_ref, bs_ref,
                       out_ref, k_ref, v_ref, *, num_segments, scale):
    i = pl.program_id(0)

    @pl.when(i == 0)
    def _init():
        x_full = x_full_ref[:]
        k = jnp.dot(x_full, wk_ref[:],
                    preferred_element_type=jnp.float32) + bk_ref[:]
        v = jnp.dot(x_full, wv_ref[:],
                    preferred_element_type=jnp.float32) + bv_ref[:]
        k_ref[:] = k.astype(jnp.bfloat16)
        v_ref[:] = v.astype(jnp.bfloat16)
        out_ref[:] = jnp.full_like(out_ref, -jnp.inf)

    x_t = x_tile_ref[:]
    q = (jnp.dot(x_t, wq_ref[:],
                 preferred_element_type=jnp.float32) + bq_ref[:]) * scale

    s = lax.dot_general(q.astype(jnp.bfloat16), k_ref[:],
                        (((1,), (1,)), ((), ())),
                        preferred_element_type=jnp.float32)
    seg_c = segc_ref[0]
    seg_r = segr_ref[:]
    mask = seg_c == seg_r
    s = jnp.where(mask, s, -jnp.inf)
    m = jnp.max(s, axis=1, keepdims=True)
    p = jnp.exp2(s - m)
    denom = jnp.sum(p, axis=1, keepdims=True)

    o = jnp.dot(p.astype(jnp.bfloat16), v_ref[:],
                preferred_element_type=jnp.float32) * (1.0 / denom)
    o = o + jnp.dot(x_t, ws_ref[:],
                    preferred_element_type=jnp.float32) + bs_ref[:]

    first = jnp.min(seg_c)
    last = jnp.max(seg_c)
    for b in range(num_segments):
        @pl.when((b >= first) & (b <= last))
        def _pool():
            mb = seg_c == b
            pb = jnp.max(jnp.where(mb, o, -jnp.inf), axis=0,
                         keepdims=True)
            out_ref[b:b + 1, :] = jnp.maximum(out_ref[b:b + 1, :], pb)


def kernel(x, subgbatch, Wq, bq, Wk, bk, Wv, bv, Wskip, bskip):
    n, d = x.shape
    c = Wq.shape[1]
    num_segments = 16
    t = _ROW_TILE
    num_tiles = n // t
    seg = subgbatch.astype(jnp.int32)
    segc = seg.reshape(num_tiles, t, 1)
    segr = seg.reshape(1, n)

    fn = pl.pallas_call(
        functools.partial(_atten_pool_kernel, num_segments=num_segments,
                          scale=math.log2(math.e) / math.sqrt(c)),
        grid=(num_tiles,),
        in_specs=[
            pl.BlockSpec((n, d), lambda i: (0, 0)),
            pl.BlockSpec((t, d), lambda i: (i, 0)),
            pl.BlockSpec((1, t, 1), lambda i: (i, 0, 0)),
            pl.BlockSpec((1, n), lambda i: (0, 0)),
            pl.BlockSpec((d, c), lambda i: (0, 0)),
            pl.BlockSpec((1, c), lambda i: (0, 0)),
            pl.BlockSpec((d, c), lambda i: (0, 0)),
            pl.BlockSpec((1, c), lambda i: (0, 0)),
            pl.BlockSpec((d, c), lambda i: (0, 0)),
            pl.BlockSpec((1, c), lambda i: (0, 0)),
            pl.BlockSpec((d, c), lambda i: (0, 0)),
            pl.BlockSpec((1, c), lambda i: (0, 0)),
        ],
        out_specs=pl.BlockSpec((num_segments, c), lambda i: (0, 0)),
        scratch_shapes=[
            pltpu.VMEM((n, c), jnp.bfloat16),
            pltpu.VMEM((n, c), jnp.bfloat16),
        ],
        out_shape=jax.ShapeDtypeStruct((num_segments, c), jnp.float32),
    )
    return fn(x, x, segc, segr,
              Wq, bq.reshape(1, c), Wk, bk.reshape(1, c),
              Wv, bv.reshape(1, c), Wskip, bskip.reshape(1, c))
